# baseline (speedup 1.0000x reference)
import functools

import jax
import jax.numpy as jnp
from jax import lax
from jax.experimental import pallas as pl
from jax.experimental.pallas import tpu as pltpu
from jax.experimental.pallas import tpu_sc as plsc

_INFO = plsc.get_sparse_core_info()
_NC, _NS, _L = _INFO.num_cores, _INFO.num_subcores, _INFO.num_lanes
_NW = _NC * _NS


def _row_lookup(t_end, m):
    mesh = plsc.VectorSubcoreMesh(
        core_axis_name="c", subcore_axis_name="s", num_cores=1, num_subcores=1
    )

    @functools.partial(
        pl.kernel,
        out_type=jax.ShapeDtypeStruct((1, m), jnp.float32),
        mesh=mesh,
        scratch_types=[
            pltpu.VMEM((_L,), jnp.int32),
        ],
    )
    def k(u_hbm, idx_hbm, out_hbm, idx_v):
        pltpu.sync_copy(idx_hbm, idx_v)
        tval = idx_v[...][0]
        pltpu.sync_copy(u_hbm.at[pl.ds(tval, 1)], out_hbm)

    return k


def kernel(u, t):
    t_end, m = u.shape
    t_arr = jnp.asarray(t, dtype=jnp.int32)
    idx = jnp.full((_L,), jnp.minimum(t_arr, t_end - 1), dtype=jnp.int32)
    out = _row_lookup(t_end, m)(u, idx)
    return out.reshape(m)


# speedup vs baseline: 1.0851x; 1.0851x over previous
import functools

import jax
import jax.numpy as jnp
from jax import lax
from jax.experimental import pallas as pl
from jax.experimental.pallas import tpu as pltpu
from jax.experimental.pallas import tpu_sc as plsc

_INFO = plsc.get_sparse_core_info()
_NC, _NS, _L = _INFO.num_cores, _INFO.num_subcores, _INFO.num_lanes
_NW = _NC * _NS


def _row_lookup(t_end, m):
    mesh = plsc.ScalarSubcoreMesh(axis_name="c", num_cores=1)

    @functools.partial(
        pl.kernel,
        out_type=jax.ShapeDtypeStruct((1, m), jnp.float32),
        mesh=mesh,
        scratch_types=[
            pltpu.SMEM((1,), jnp.int32),
        ],
    )
    def k(u_hbm, idx_hbm, out_hbm, idx_s):
        pltpu.sync_copy(idx_hbm, idx_s)
        tval = idx_s[0]
        pltpu.sync_copy(u_hbm.at[pl.ds(tval, 1)], out_hbm)

    return k


def kernel(u, t):
    t_end, m = u.shape
    t_arr = jnp.asarray(t, dtype=jnp.int32)
    idx = jnp.minimum(t_arr, t_end - 1).reshape(1)
    out = _row_lookup(t_end, m)(u, idx)
    return out.reshape(m)


# speedup vs baseline: 1.0954x; 1.0095x over previous
import functools

import jax
import jax.numpy as jnp
from jax.experimental import pallas as pl
from jax.experimental.pallas import tpu as pltpu
from jax.experimental.pallas import tpu_sc as plsc


def _row_lookup(t_end, m):
    mesh = plsc.ScalarSubcoreMesh(axis_name="c", num_cores=1)

    @functools.partial(
        pl.kernel,
        out_type=jax.ShapeDtypeStruct((1, m), jnp.float32),
        mesh=mesh,
        scratch_types=[
            pltpu.SMEM((1,), jnp.int32),
        ],
    )
    def k(u_hbm, t_hbm, zero_hbm, out_hbm, idx_s):
        pltpu.sync_copy(t_hbm, idx_s)
        tval = idx_s[0]
        safe = jnp.minimum(tval, t_end - 1)

        @pl.when(tval < t_end)
        def _in_range():
            pltpu.sync_copy(u_hbm.at[pl.ds(safe, 1)], out_hbm)

        @pl.when(tval >= t_end)
        def _out_of_range():
            pltpu.sync_copy(zero_hbm, out_hbm)

    return k


def kernel(u, t):
    t_end, m = u.shape
    t_arr = jnp.asarray(t, dtype=jnp.int32).reshape(1)
    zero_row = jnp.zeros((1, m), dtype=jnp.float32)
    out = _row_lookup(t_end, m)(u, t_arr, zero_row)
    return out.reshape(m)
